# Initial kernel scaffold; baseline (speedup 1.0000x reference)
import jax
import jax.numpy as jnp
from jax.experimental import pallas as pl


def kernel(x, weight, bias):
    raise NotImplementedError("write your pallas kernel here")


# speedup vs baseline: 1.3541x; 1.3541x over previous
import math

import jax
import jax.numpy as jnp
import numpy as np
from jax.experimental import pallas as pl
from jax.experimental.pallas import tpu as pltpu

_IN_SHAPE = (96, 96, 96)
_KERNEL_SHAPE = (2, 4, 4)
_DOMAIN = (2.0, 2.0, 2.0)
_KSIZE = (_KERNEL_SHAPE[0] - 1) * _KERNEL_SHAPE[1] * _KERNEL_SHAPE[2] + 1


def _disco_psi():
    nr, nt, nph = _KERNEL_SHAPE
    hs = [_DOMAIN[i] / _IN_SHAPE[i] for i in range(3)]
    r_cutoff = 1.5 * max(hs)
    ps = [int(np.floor(r_cutoff / hs[i])) for i in range(3)]
    zz = np.arange(-ps[0], ps[0] + 1) * hs[0]
    yy = np.arange(-ps[1], ps[1] + 1) * hs[1]
    xx = np.arange(-ps[2], ps[2] + 1) * hs[2]
    dz, dy, dx = np.meshgrid(zz, yy, xx, indexing='ij')
    r = np.sqrt(dz ** 2 + dy ** 2 + dx ** 2 + 1e-12)
    phi = np.arctan2(dy, dx)
    theta = np.arccos(np.clip(dz / (r + 1e-12), -1.0, 1.0))
    mask = (r <= r_cutoff + 1e-9).astype(np.float64)
    psi = np.zeros((_KSIZE,) + r.shape, dtype=np.float64)
    dr = r_cutoff / (nr - 1) if nr > 1 else r_cutoff
    psi[0] = np.clip(1.0 - r / dr, 0.0, None) * mask
    dtheta = math.pi / nt
    dphi = 2.0 * math.pi / nph
    ik = 1
    for ir in range(1, nr):
        rad = np.clip(1.0 - np.abs(r - ir * dr) / dr, 0.0, None)
        for it in range(nt):
            tc = (it + 0.5) * dtheta
            tv = np.clip(1.0 - np.abs(theta - tc) / dtheta, 0.0, None)
            for ip in range(nph):
                pc = -math.pi + (ip + 0.5) * dphi
                d = np.abs(phi - pc)
                d = np.minimum(d, 2.0 * math.pi - d)
                pv = np.clip(1.0 - d / dphi, 0.0, None)
                psi[ik] = rad * tv * pv * mask
                ik += 1
    q = hs[0] * hs[1] * hs[2]
    for k in range(_KSIZE):
        s = psi[k].sum() * q
        psi[k] = psi[k] / (s + 1e-9)
    return psi.astype(np.float32)


_PSI_FLAT = jnp.asarray(_disco_psi().reshape(_KSIZE, -1))


def _contract_kernel_body(w_ref, psi_ref, out_ref):
    out_ref[...] = jnp.dot(w_ref[...], psi_ref[...],
                           preferred_element_type=jnp.float32)


def _conv_body(wt_ref, b_ref, xp_ref, xc_ref, xn_ref, out_ref, p_ref):
    C, Y, X = xp_ref.shape[0], xp_ref.shape[2], xp_ref.shape[3]
    zi = pl.program_id(0)
    yi = pl.program_id(1)
    nz = pl.num_programs(0)
    ny = pl.num_programs(1)
    y0 = yi * 8 - 1

    zero_row = jnp.zeros((C, 1, X), dtype=jnp.float32)
    slabs = []
    for a, ref in enumerate((xp_ref, xc_ref, xn_ref)):
        s = jnp.clip(y0, 0, Y - 10)
        rows = ref[:, 0, pl.ds(s, 10), :]
        rows_first = jnp.concatenate([zero_row, rows[:, :9]], axis=1)
        rows_last = jnp.concatenate([rows[:, 1:], zero_row], axis=1)
        rows = jnp.where(yi == 0, rows_first,
                         jnp.where(yi == ny - 1, rows_last, rows))
        if a != 1:
            zok = jnp.logical_and(zi + a - 1 >= 0, zi + a - 1 <= nz - 1)
            rows = jnp.where(zok, rows, 0.0)
        slabs.append(rows)

    zcol = jnp.zeros((C, 1), dtype=jnp.float32)
    for a in range(3):
        for yw in range(10):
            row = slabs[a][:, yw]
            left = jnp.concatenate([zcol, row[:, :X - 1]], axis=1)
            right = jnp.concatenate([row[:, 1:], zcol], axis=1)
            base = ((a * 10 + yw) * 3) * C
            p_ref[pl.ds(base, C), :] = left
            p_ref[pl.ds(base + C, C), :] = row
            p_ref[pl.ds(base + 2 * C, C), :] = right

    acc = jnp.dot(wt_ref[...], p_ref[...], preferred_element_type=jnp.float32)
    acc = acc + b_ref[:, :X]
    out_ref[...] = acc.reshape(out_ref.shape)


def kernel(x, weight, bias):
    x3 = x[0]
    C, Z, Y, X = x3.shape
    O, _, K = weight.shape

    w2 = weight.reshape(O * C, K)
    k5flat = pl.pallas_call(
        _contract_kernel_body,
        out_shape=jax.ShapeDtypeStruct((O * C, 27), jnp.float32),
    )(w2, _PSI_FLAT)
    k5 = k5flat.reshape(O, C, 3, 3, 3)

    k5t = k5.transpose(0, 2, 3, 4, 1)
    wt = jnp.stack(
        [jnp.pad(k5t, ((0, 0), (0, 0), (jy, 7 - jy), (0, 0), (0, 0)))
         for jy in range(8)], axis=1)
    wt = wt.reshape(O * 8, 3 * 10 * 3 * C)

    bias_m = jnp.broadcast_to(jnp.repeat(bias, 8)[:, None], (O * 8, 128))

    grid = (Z, Y // 8)
    out = pl.pallas_call(
        _conv_body,
        grid=grid,
        in_specs=[
            pl.BlockSpec((O * 8, 3 * 10 * 3 * C), lambda zi, yi: (0, 0)),
            pl.BlockSpec((O * 8, 128), lambda zi, yi: (0, 0)),
            pl.BlockSpec((C, 1, Y, X),
                         lambda zi, yi: (0, jnp.maximum(zi - 1, 0), 0, 0)),
            pl.BlockSpec((C, 1, Y, X), lambda zi, yi: (0, zi, 0, 0)),
            pl.BlockSpec((C, 1, Y, X),
                         lambda zi, yi: (0, jnp.minimum(zi + 1, Z - 1), 0, 0)),
        ],
        out_specs=pl.BlockSpec((O, 1, 8, X), lambda zi, yi: (0, zi, yi, 0)),
        out_shape=jax.ShapeDtypeStruct((O, Z, Y, X), jnp.float32),
        scratch_shapes=[pltpu.VMEM((3 * 10 * 3 * C, X), jnp.float32)],
    )(wt, bias_m, x3, x3, x3)

    return out[None]


# speedup vs baseline: 4.0268x; 2.9736x over previous
import math

import jax
import jax.numpy as jnp
import numpy as np
from jax.experimental import pallas as pl
from jax.experimental.pallas import tpu as pltpu

_IN_SHAPE = (96, 96, 96)
_KERNEL_SHAPE = (2, 4, 4)
_DOMAIN = (2.0, 2.0, 2.0)
_KSIZE = (_KERNEL_SHAPE[0] - 1) * _KERNEL_SHAPE[1] * _KERNEL_SHAPE[2] + 1


def _disco_psi():
    nr, nt, nph = _KERNEL_SHAPE
    hs = [_DOMAIN[i] / _IN_SHAPE[i] for i in range(3)]
    r_cutoff = 1.5 * max(hs)
    ps = [int(np.floor(r_cutoff / hs[i])) for i in range(3)]
    zz = np.arange(-ps[0], ps[0] + 1) * hs[0]
    yy = np.arange(-ps[1], ps[1] + 1) * hs[1]
    xx = np.arange(-ps[2], ps[2] + 1) * hs[2]
    dz, dy, dx = np.meshgrid(zz, yy, xx, indexing='ij')
    r = np.sqrt(dz ** 2 + dy ** 2 + dx ** 2 + 1e-12)
    phi = np.arctan2(dy, dx)
    theta = np.arccos(np.clip(dz / (r + 1e-12), -1.0, 1.0))
    mask = (r <= r_cutoff + 1e-9).astype(np.float64)
    psi = np.zeros((_KSIZE,) + r.shape, dtype=np.float64)
    dr = r_cutoff / (nr - 1) if nr > 1 else r_cutoff
    psi[0] = np.clip(1.0 - r / dr, 0.0, None) * mask
    dtheta = math.pi / nt
    dphi = 2.0 * math.pi / nph
    ik = 1
    for ir in range(1, nr):
        rad = np.clip(1.0 - np.abs(r - ir * dr) / dr, 0.0, None)
        for it in range(nt):
            tc = (it + 0.5) * dtheta
            tv = np.clip(1.0 - np.abs(theta - tc) / dtheta, 0.0, None)
            for ip in range(nph):
                pc = -math.pi + (ip + 0.5) * dphi
                d = np.abs(phi - pc)
                d = np.minimum(d, 2.0 * math.pi - d)
                pv = np.clip(1.0 - d / dphi, 0.0, None)
                psi[ik] = rad * tv * pv * mask
                ik += 1
    q = hs[0] * hs[1] * hs[2]
    for k in range(_KSIZE):
        s = psi[k].sum() * q
        psi[k] = psi[k] / (s + 1e-9)
    return psi.astype(np.float32)


_PSI_FLAT_NP = _disco_psi().reshape(_KSIZE, -1)


def _contract_kernel_body(w_ref, psi_ref, out_ref):
    out_ref[...] = jnp.dot(w_ref[...], psi_ref[...],
                           preferred_element_type=jnp.float32)


def _conv_body(wt_ref, b_ref, xmid_ref, xh1_ref, xh2_ref, out_ref, p_ref):
    C, X = xmid_ref.shape[2], xmid_ref.shape[3]
    O = out_ref.shape[0]
    yt = pl.program_id(1)
    y0 = yt * 8

    for zz in range(8):
        for a in range(3):
            zoff = zz + a
            if zoff <= 7:
                chunk = xmid_ref[zoff, pl.ds(y0, 10), :, :]
            elif zoff == 8:
                chunk = xh1_ref[0, pl.ds(y0, 10), :, :]
            else:
                chunk = xh2_ref[0, pl.ds(y0, 10), :, :]
            p_ref[pl.ds(a * 10 * C, 10 * C), :] = chunk.reshape(10 * C, X)

        p = p_ref[...]
        o0 = jnp.dot(wt_ref[0], p, preferred_element_type=jnp.float32)
        o1 = jnp.dot(wt_ref[1], p, preferred_element_type=jnp.float32)
        o2 = jnp.dot(wt_ref[2], p, preferred_element_type=jnp.float32)
        zc = jnp.zeros((o1.shape[0], 1), dtype=jnp.float32)
        acc = o1 + jnp.concatenate([zc, o0[:, :X - 1]], axis=1)
        acc = acc + jnp.concatenate([o2[:, 1:], zc], axis=1)
        acc = acc + b_ref[:, :X]
        out_ref[:, 0, zz, 0, :, :] = acc.reshape(O, 8, X)


def kernel(x, weight, bias):
    x3 = x[0]
    C, Z, Y, X = x3.shape
    O, _, K = weight.shape

    w2 = weight.reshape(O * C, K)
    k5flat = pl.pallas_call(
        _contract_kernel_body,
        out_shape=jax.ShapeDtypeStruct((O * C, 27), jnp.float32),
    )(w2, jnp.asarray(_PSI_FLAT_NP))
    k5 = k5flat.reshape(O, C, 3, 3, 3)

    k5t = k5.transpose(4, 0, 2, 3, 1)
    wt = jnp.stack(
        [jnp.pad(k5t, ((0, 0), (0, 0), (0, 0), (jy, 7 - jy), (0, 0)))
         for jy in range(8)], axis=2)
    wt = wt.reshape(3, O * 8, 3 * 10 * C)

    bias_m = jnp.broadcast_to(jnp.repeat(bias, 8)[:, None], (O * 8, 128))
    wtb = wt.astype(jnp.bfloat16)

    xt = jnp.transpose(x3.astype(jnp.bfloat16), (1, 2, 0, 3))
    xtp = jnp.pad(xt, ((1, 1), (1, 7), (0, 0), (0, 0)))
    Yp = Y + 8

    grid = (Z // 8, Y // 8)
    out6 = pl.pallas_call(
        _conv_body,
        grid=grid,
        in_specs=[
            pl.BlockSpec((3, O * 8, 3 * 10 * C), lambda zt, yt: (0, 0, 0)),
            pl.BlockSpec((O * 8, 128), lambda zt, yt: (0, 0)),
            pl.BlockSpec((8, Yp, C, X), lambda zt, yt: (zt, 0, 0, 0)),
            pl.BlockSpec((1, Yp, C, X), lambda zt, yt: (8 * zt + 8, 0, 0, 0)),
            pl.BlockSpec((1, Yp, C, X), lambda zt, yt: (8 * zt + 9, 0, 0, 0)),
        ],
        out_specs=pl.BlockSpec((O, 1, 8, 1, 8, X),
                               lambda zt, yt: (0, zt, 0, yt, 0, 0)),
        out_shape=jax.ShapeDtypeStruct((O, Z // 8, 8, Y // 8, 8, X),
                                       jnp.float32),
        scratch_shapes=[pltpu.VMEM((3 * 10 * C, X), jnp.bfloat16)],
    )(wtb, bias_m, xtp, xtp, xtp)

    return out6.reshape(O, Z, Y, X)[None]


# speedup vs baseline: 4.3248x; 1.0740x over previous
import math

import jax
import jax.numpy as jnp
import numpy as np
from jax.experimental import pallas as pl
from jax.experimental.pallas import tpu as pltpu

_IN_SHAPE = (96, 96, 96)
_KERNEL_SHAPE = (2, 4, 4)
_DOMAIN = (2.0, 2.0, 2.0)
_KSIZE = (_KERNEL_SHAPE[0] - 1) * _KERNEL_SHAPE[1] * _KERNEL_SHAPE[2] + 1


def _disco_psi():
    nr, nt, nph = _KERNEL_SHAPE
    hs = [_DOMAIN[i] / _IN_SHAPE[i] for i in range(3)]
    r_cutoff = 1.5 * max(hs)
    ps = [int(np.floor(r_cutoff / hs[i])) for i in range(3)]
    zz = np.arange(-ps[0], ps[0] + 1) * hs[0]
    yy = np.arange(-ps[1], ps[1] + 1) * hs[1]
    xx = np.arange(-ps[2], ps[2] + 1) * hs[2]
    dz, dy, dx = np.meshgrid(zz, yy, xx, indexing='ij')
    r = np.sqrt(dz ** 2 + dy ** 2 + dx ** 2 + 1e-12)
    phi = np.arctan2(dy, dx)
    theta = np.arccos(np.clip(dz / (r + 1e-12), -1.0, 1.0))
    mask = (r <= r_cutoff + 1e-9).astype(np.float64)
    psi = np.zeros((_KSIZE,) + r.shape, dtype=np.float64)
    dr = r_cutoff / (nr - 1) if nr > 1 else r_cutoff
    psi[0] = np.clip(1.0 - r / dr, 0.0, None) * mask
    dtheta = math.pi / nt
    dphi = 2.0 * math.pi / nph
    ik = 1
    for ir in range(1, nr):
        rad = np.clip(1.0 - np.abs(r - ir * dr) / dr, 0.0, None)
        for it in range(nt):
            tc = (it + 0.5) * dtheta
            tv = np.clip(1.0 - np.abs(theta - tc) / dtheta, 0.0, None)
            for ip in range(nph):
                pc = -math.pi + (ip + 0.5) * dphi
                d = np.abs(phi - pc)
                d = np.minimum(d, 2.0 * math.pi - d)
                pv = np.clip(1.0 - d / dphi, 0.0, None)
                psi[ik] = rad * tv * pv * mask
                ik += 1
    q = hs[0] * hs[1] * hs[2]
    for k in range(_KSIZE):
        s = psi[k].sum() * q
        psi[k] = psi[k] / (s + 1e-9)
    return psi.astype(np.float32)


_PSI_FLAT_NP = _disco_psi().reshape(_KSIZE, -1)


def _contract_kernel_body(w_ref, psi_ref, out_ref):
    out_ref[...] = jnp.dot(w_ref[...], psi_ref[...],
                           preferred_element_type=jnp.float32)


def _conv_body(wt_ref, b_ref, xmid_ref, zprev_ref, znext_ref, out_ref,
               p_ref, s_ref):
    C, Y, X = xmid_ref.shape[0], xmid_ref.shape[2], xmid_ref.shape[3]
    Yp = s_ref.shape[2]
    O = out_ref.shape[0]
    zt = pl.program_id(0)
    yt = pl.program_id(1)
    nz = pl.num_programs(0)
    y0 = yt * 8

    @pl.when(jnp.logical_and(zt == 0, yt == 0))
    def _init_borders():
        s_ref[:, :, 0, :] = jnp.zeros((C, 10, X), jnp.bfloat16)
        s_ref[:, :, pl.ds(Y + 1, 7), :] = jnp.zeros((C, 10, 7, X),
                                                    jnp.bfloat16)

    @pl.when(yt == 0)
    def _stage():
        s_ref[:, 0, pl.ds(1, Y), :] = zprev_ref[:, 0].astype(jnp.bfloat16)
        s_ref[:, pl.ds(1, 8), pl.ds(1, Y), :] = (
            xmid_ref[...].astype(jnp.bfloat16))
        s_ref[:, 9, pl.ds(1, Y), :] = znext_ref[:, 0].astype(jnp.bfloat16)

    @pl.when(jnp.logical_and(yt == 0, zt == 0))
    def _zero_zlo():
        s_ref[:, 0, :, :] = jnp.zeros((C, Yp, X), jnp.bfloat16)

    @pl.when(jnp.logical_and(yt == 0, zt == nz - 1))
    def _zero_zhi():
        s_ref[:, 9, :, :] = jnp.zeros((C, Yp, X), jnp.bfloat16)

    for zz in range(8):
        for a in range(3):
            chunk = s_ref[:, zz + a, pl.ds(y0, 10), :]
            p_ref[pl.ds(a * C * 10, C * 10), :] = chunk.reshape(C * 10, X)

        p = p_ref[...]
        o0 = jnp.dot(wt_ref[0], p, preferred_element_type=jnp.float32)
        o1 = jnp.dot(wt_ref[1], p, preferred_element_type=jnp.float32)
        o2 = jnp.dot(wt_ref[2], p, preferred_element_type=jnp.float32)
        zc = jnp.zeros((o1.shape[0], 1), dtype=jnp.float32)
        acc = o1 + jnp.concatenate([zc, o0[:, :X - 1]], axis=1)
        acc = acc + jnp.concatenate([o2[:, 1:], zc], axis=1)
        acc = acc + b_ref[:, :X]
        out_ref[:, 0, zz, 0, :, :] = acc.reshape(O, 8, X)


def kernel(x, weight, bias):
    x3 = x[0]
    C, Z, Y, X = x3.shape
    O, _, K = weight.shape

    w2 = weight.reshape(O * C, K)
    k5flat = pl.pallas_call(
        _contract_kernel_body,
        out_shape=jax.ShapeDtypeStruct((O * C, 27), jnp.float32),
    )(w2, jnp.asarray(_PSI_FLAT_NP))
    k5 = k5flat.reshape(O, C, 3, 3, 3)

    k5t = k5.transpose(4, 0, 2, 1, 3)
    wt = jnp.stack(
        [jnp.pad(k5t, ((0, 0), (0, 0), (0, 0), (0, 0), (jy, 7 - jy)))
         for jy in range(8)], axis=2)
    wt = wt.reshape(3, O * 8, 3 * C * 10)

    bias_m = jnp.broadcast_to(jnp.repeat(bias, 8)[:, None], (O * 8, 128))
    wtb = wt.astype(jnp.bfloat16)

    grid = (Z // 8, Y // 8)
    out6 = pl.pallas_call(
        _conv_body,
        grid=grid,
        in_specs=[
            pl.BlockSpec((3, O * 8, 3 * C * 10), lambda zt, yt: (0, 0, 0)),
            pl.BlockSpec((O * 8, 128), lambda zt, yt: (0, 0)),
            pl.BlockSpec((C, 8, Y, X), lambda zt, yt: (0, zt, 0, 0)),
            pl.BlockSpec((C, 1, Y, X),
                         lambda zt, yt: (0, jnp.maximum(8 * zt - 1, 0), 0, 0)),
            pl.BlockSpec((C, 1, Y, X),
                         lambda zt, yt: (0, jnp.minimum(8 * zt + 8, Z - 1),
                                         0, 0)),
        ],
        out_specs=pl.BlockSpec((O, 1, 8, 1, 8, X),
                               lambda zt, yt: (0, zt, 0, yt, 0, 0)),
        out_shape=jax.ShapeDtypeStruct((O, Z // 8, 8, Y // 8, 8, X),
                                       jnp.float32),
        scratch_shapes=[pltpu.VMEM((3 * C * 10, X), jnp.bfloat16),
                        pltpu.VMEM((C, 10, Y + 8, X), jnp.bfloat16)],
    )(wtb, bias_m, x3, x3, x3)

    return out6.reshape(O, Z, Y, X)[None]
